# baseline (speedup 1.0000x reference)
import jax
import jax.numpy as jnp
from jax.experimental import pallas as pl

_K = 1000
_KPAD = 1024
_BM = 1024


def _kmeans_block(x_ref, c_ref, dist_ref, label_ref):
    xb = x_ref[...]
    c = c_ref[...]
    a2 = jnp.sum(xb * xb, axis=1, keepdims=True)
    b2 = jnp.sum(c * c, axis=1)
    xc = jax.lax.dot_general(
        xb, c, (((1,), (1,)), ((), ())),
        preferred_element_type=jnp.float32)
    d2 = a2 + b2[None, :] - 2.0 * xc
    col = jax.lax.broadcasted_iota(jnp.int32, d2.shape, 1)
    d2 = jnp.where(col < _K, jnp.maximum(d2, 1e-12), jnp.inf)
    dist_ref[...] = jnp.sqrt(jnp.min(d2, axis=1))
    label_ref[...] = jnp.argmin(d2, axis=1).astype(jnp.int32)


@jax.jit
def kernel(x, centers):
    n = x.shape[0]
    c_pad = jnp.zeros((_KPAD, centers.shape[1]), centers.dtype)
    c_pad = c_pad.at[:_K].set(centers)
    grid = (n // _BM,)
    dist, labels = pl.pallas_call(
        _kmeans_block,
        grid=grid,
        in_specs=[
            pl.BlockSpec((_BM, x.shape[1]), lambda i: (i, 0)),
            pl.BlockSpec((_KPAD, centers.shape[1]), lambda i: (0, 0)),
        ],
        out_specs=[
            pl.BlockSpec((_BM,), lambda i: (i,)),
            pl.BlockSpec((_BM,), lambda i: (i,)),
        ],
        out_shape=[
            jax.ShapeDtypeStruct((n,), jnp.float32),
            jax.ShapeDtypeStruct((n,), jnp.int32),
        ],
    )(x, c_pad)
    return dist, labels


# speedup vs baseline: 1.0780x; 1.0780x over previous
import jax
import jax.numpy as jnp
from jax.experimental import pallas as pl

_K = 1000
_KPAD = 1024
_BM = 1024


def _kmeans_block(x_ref, c_ref, dist_ref, label_ref):
    xb = x_ref[...]
    c = c_ref[...]
    a2 = jnp.sum(xb * xb, axis=1)
    idx = jax.lax.broadcasted_iota(jnp.int32, (_KPAD,), 0)
    b2 = jnp.where(idx < _K, jnp.sum(c * c, axis=1), jnp.inf)
    xc = jax.lax.dot_general(
        xb, c * -2.0, (((1,), (1,)), ((), ())),
        preferred_element_type=jnp.float32)
    t = xc + b2[None, :]
    m = jnp.min(t, axis=1)
    label_ref[...] = jnp.argmin(t, axis=1).astype(jnp.int32)
    dist_ref[...] = jnp.sqrt(jnp.maximum(m + a2, 1e-12))


@jax.jit
def kernel(x, centers):
    n = x.shape[0]
    c_pad = jnp.zeros((_KPAD, centers.shape[1]), centers.dtype)
    c_pad = c_pad.at[:_K].set(centers)
    grid = (n // _BM,)
    dist, labels = pl.pallas_call(
        _kmeans_block,
        grid=grid,
        in_specs=[
            pl.BlockSpec((_BM, x.shape[1]), lambda i: (i, 0)),
            pl.BlockSpec((_KPAD, centers.shape[1]), lambda i: (0, 0)),
        ],
        out_specs=[
            pl.BlockSpec((_BM,), lambda i: (i,)),
            pl.BlockSpec((_BM,), lambda i: (i,)),
        ],
        out_shape=[
            jax.ShapeDtypeStruct((n,), jnp.float32),
            jax.ShapeDtypeStruct((n,), jnp.int32),
        ],
    )(x, c_pad)
    return dist, labels
